# baseline (speedup 1.0000x reference)
import functools

import jax
import jax.numpy as jnp
from jax.experimental import pallas as pl
from jax.experimental.pallas import tpu as pltpu
from jax.experimental.pallas import tpu_sc as plsc

E = 64
D = 768
FF = 1024
N = 2048
T = 128
TM = N // T
STEPS = TM + E - 1
GT = 128
GM = N // GT
EPAD = 128
_BIG = 1 << 30


def _gating_body(x_ref, gwt_ref, eid_ref, grank_ref, cnt_ref):
    t = pl.program_id(0)

    @pl.when(t == 0)
    def _():
        cnt_ref[...] = jnp.zeros_like(cnt_ref)

    xb = x_ref[...]
    s = jnp.dot(xb, gwt_ref[...], preferred_element_type=jnp.float32)
    col = jax.lax.broadcasted_iota(jnp.int32, s.shape, 1)
    s = jnp.where(col < E, s, -jnp.inf)
    m = jnp.max(s, axis=1, keepdims=True)
    idx = jnp.min(jnp.where(s == m, col, E), axis=1)
    onehot = (col == idx[:, None]).astype(jnp.int32)
    ri = jax.lax.broadcasted_iota(jnp.int32, (GT, GT), 0)
    rj = jax.lax.broadcasted_iota(jnp.int32, (GT, GT), 1)
    same = (idx[:, None] == idx[None, :]) & (rj < ri)
    rank = jnp.sum(same.astype(jnp.int32), axis=1)
    base = jnp.sum(onehot * cnt_ref[...], axis=1)
    eid_ref[...] = idx.reshape(1, 1, GT)
    grank_ref[...] = (base + rank).reshape(1, 1, GT)
    cnt_ref[...] += jnp.sum(onehot, axis=0, keepdims=True)


F2 = 2
FS = FF // F2


def _ffn_body(st_ref, se_ref, sv_ref, off_ref,
              x_ref, w1_ref, w2_ref, gwt_ref, out_ref, w_ref):
    s = pl.program_id(0)
    f = pl.program_id(1)
    t = st_ref[s]
    e = se_ref[s]
    valid = sv_ref[s]
    prev_t = st_ref[jnp.maximum(s - 1, 0)]
    first = (f == 0) & jnp.logical_or(s == 0, prev_t != t)

    @pl.when(first)
    def _():
        out_ref[...] = jnp.zeros_like(out_ref)

    xb = x_ref[...]

    @pl.when(f == 0)
    def _():
        sc = jnp.dot(xb, gwt_ref[...], preferred_element_type=jnp.float32)
        col = jax.lax.broadcasted_iota(jnp.int32, sc.shape, 1)
        sc = jnp.where(col < E, sc, -jnp.inf)
        m = jnp.max(sc, axis=1, keepdims=True)
        w_ref[...] = 1.0 / jnp.sum(jnp.exp(sc - m), axis=1, keepdims=True)

    h = jnp.dot(xb, w1_ref[0], preferred_element_type=jnp.float32)
    h = h * jax.nn.sigmoid(h)
    y = jnp.dot(h, w2_ref[0], preferred_element_type=jnp.float32)

    rows = t * T + jax.lax.broadcasted_iota(jnp.int32, (T, 1), 0)
    mask = (rows >= off_ref[e]) & (rows < off_ref[e + 1]) & (valid > 0)
    out_ref[...] += jnp.where(mask, y * w_ref[...], 0.0)


def _route_metadata(eid, grank, counts):
    eid = eid.astype(jnp.int32)
    offsets = jnp.concatenate(
        [jnp.zeros((1,), jnp.int32), jnp.cumsum(counts).astype(jnp.int32)]
    )
    inv_idx = (offsets[eid] + grank).astype(jnp.int32)
    sort_idx = (
        jnp.zeros((N,), jnp.int32)
        .at[inv_idx]
        .set(jnp.arange(N, dtype=jnp.int32))
    )

    t = jnp.arange(TM, dtype=jnp.int32)[:, None]
    e = jnp.arange(E, dtype=jnp.int32)[None, :]
    lo = offsets[:-1][None, :]
    hi = offsets[1:][None, :]
    valid = (lo < (t + 1) * T) & (hi > t * T) & (hi > lo)

    vflat = valid.reshape(-1)
    keys = (t * E + e).reshape(-1)
    cum = jnp.cumsum(vflat.astype(jnp.int32))
    nv = cum[-1]
    pos_step = jnp.where(vflat, cum - 1, STEPS)
    scat = jnp.zeros((STEPS,), jnp.int32).at[pos_step].set(keys)
    last_key = jnp.take(scat, nv - 1)
    sidx = jnp.arange(STEPS, dtype=jnp.int32)
    key_s = jnp.where(sidx < nv, scat, last_key)
    step_t = key_s // E
    step_e = key_s % E
    step_v = (sidx < nv).astype(jnp.int32)
    return sort_idx, inv_idx, offsets, step_t, step_e, step_v


def _sc_row_gather(table, idx):
    info = plsc.get_sparse_core_info()
    nw = info.num_cores * info.num_subcores
    bpw = N // nw
    mesh = plsc.VectorSubcoreMesh(core_axis_name="c", subcore_axis_name="s")

    @functools.partial(
        pl.kernel,
        mesh=mesh,
        out_type=jax.ShapeDtypeStruct((N, D), jnp.float32),
        scratch_types=[
            pltpu.VMEM((bpw,), jnp.int32),
            pltpu.VMEM((bpw, D), jnp.float32),
            pltpu.SemaphoreType.DMA,
        ],
    )
    def gather_k(table_hbm, idx_hbm, out_hbm, idx_v, rows_v, sem):
        wid = jax.lax.axis_index("s") * info.num_cores + jax.lax.axis_index("c")
        base = wid * bpw
        pltpu.sync_copy(idx_hbm.at[pl.ds(base, bpw)], idx_v)
        pltpu.async_copy(table_hbm.at[idx_v], rows_v, sem).wait()
        pltpu.sync_copy(rows_v, out_hbm.at[pl.ds(base, bpw)])

    return gather_k(table, idx)


def kernel(x, gate_w, w1, w2):
    orig_shape = x.shape
    xf = x.reshape(-1, x.shape[-1]).astype(jnp.float32)
    gwt = jnp.zeros((D, EPAD), jnp.float32).at[:, :E].set(gate_w.T)

    eid3, grank3, cnt = pl.pallas_call(
        _gating_body,
        grid=(GM,),
        in_specs=[
            pl.BlockSpec((GT, D), lambda t: (t, 0)),
            pl.BlockSpec((D, EPAD), lambda t: (0, 0)),
        ],
        out_specs=[
            pl.BlockSpec((1, 1, GT), lambda t: (t, 0, 0)),
            pl.BlockSpec((1, 1, GT), lambda t: (t, 0, 0)),
            pl.BlockSpec((1, EPAD), lambda t: (0, 0)),
        ],
        out_shape=[
            jax.ShapeDtypeStruct((GM, 1, GT), jnp.int32),
            jax.ShapeDtypeStruct((GM, 1, GT), jnp.int32),
            jax.ShapeDtypeStruct((1, EPAD), jnp.int32),
        ],
    )(xf, gwt)
    eid = eid3.reshape(N)
    grank = grank3.reshape(N)
    counts = cnt[0, :E]

    sort_idx, inv_idx, offsets, step_t, step_e, step_v = _route_metadata(
        eid, grank, counts)

    x_sorted = _sc_row_gather(xf, sort_idx)

    grid_spec = pltpu.PrefetchScalarGridSpec(
        num_scalar_prefetch=4,
        grid=(STEPS, F2),
        in_specs=[
            pl.BlockSpec((T, D), lambda s, f, st, se, sv, off: (st[s], 0)),
            pl.BlockSpec((1, D, FS),
                         lambda s, f, st, se, sv, off: (se[s], 0, f)),
            pl.BlockSpec((1, FS, D),
                         lambda s, f, st, se, sv, off: (se[s], f, 0)),
            pl.BlockSpec((D, EPAD), lambda s, f, st, se, sv, off: (0, 0)),
        ],
        out_specs=pl.BlockSpec((T, D), lambda s, f, st, se, sv, off: (st[s], 0)),
        scratch_shapes=[pltpu.VMEM((T, 1), jnp.float32)],
    )
    out_sorted = pl.pallas_call(
        _ffn_body,
        grid_spec=grid_spec,
        out_shape=jax.ShapeDtypeStruct((N, D), jnp.float32),
        compiler_params=pltpu.CompilerParams(
            dimension_semantics=("arbitrary", "arbitrary"),
        ),
    )(step_t, step_e, step_v, offsets, x_sorted, w1, w2, gwt)

    y = _sc_row_gather(out_sorted, inv_idx)
    return y.reshape(orig_shape)


# speedup vs baseline: 1.3764x; 1.3764x over previous
import functools

import jax
import jax.numpy as jnp
from jax.experimental import pallas as pl
from jax.experimental.pallas import tpu as pltpu
from jax.experimental.pallas import tpu_sc as plsc

E = 64
D = 768
FF = 1024
N = 2048
T = 256
TM = N // T
STEPS = TM + E - 1
GT = 128
GM = N // GT
EPAD = 128
_BIG = 1 << 30


def _gating_body(x_ref, gwt_ref, eid_ref, grank_ref, cnt_ref):
    t = pl.program_id(0)

    @pl.when(t == 0)
    def _():
        cnt_ref[...] = jnp.zeros_like(cnt_ref)

    xb = x_ref[...]
    s = jnp.dot(xb, gwt_ref[...], preferred_element_type=jnp.float32)
    col = jax.lax.broadcasted_iota(jnp.int32, s.shape, 1)
    s = jnp.where(col < E, s, -jnp.inf)
    m = jnp.max(s, axis=1, keepdims=True)
    idx = jnp.min(jnp.where(s == m, col, E), axis=1)
    onehot = (col == idx[:, None]).astype(jnp.int32)
    ri = jax.lax.broadcasted_iota(jnp.int32, (GT, GT), 0)
    rj = jax.lax.broadcasted_iota(jnp.int32, (GT, GT), 1)
    same = (idx[:, None] == idx[None, :]) & (rj < ri)
    rank = jnp.sum(same.astype(jnp.int32), axis=1)
    base = jnp.sum(onehot * cnt_ref[...], axis=1)
    eid_ref[...] = idx.reshape(1, 1, GT)
    grank_ref[...] = (base + rank).reshape(1, 1, GT)
    cnt_ref[...] += jnp.sum(onehot, axis=0, keepdims=True)


F2 = 1
FS = FF // F2


def _ffn_body(st_ref, se_ref, sv_ref, off_ref,
              x_ref, w1_ref, w2_ref, gwt_ref, out_ref, w_ref):
    s = pl.program_id(0)
    f = pl.program_id(1)
    t = st_ref[s]
    e = se_ref[s]
    valid = sv_ref[s]
    prev_t = st_ref[jnp.maximum(s - 1, 0)]
    first = (f == 0) & jnp.logical_or(s == 0, prev_t != t)

    @pl.when(first)
    def _():
        out_ref[...] = jnp.zeros_like(out_ref)

    xb = x_ref[...]

    @pl.when(f == 0)
    def _():
        sc = jnp.dot(xb, gwt_ref[...], preferred_element_type=jnp.float32)
        col = jax.lax.broadcasted_iota(jnp.int32, sc.shape, 1)
        sc = jnp.where(col < E, sc, -jnp.inf)
        m = jnp.max(sc, axis=1, keepdims=True)
        w_ref[...] = 1.0 / jnp.sum(jnp.exp(sc - m), axis=1, keepdims=True)

    h = jnp.dot(xb, w1_ref[0], preferred_element_type=jnp.float32)
    h = h * jax.nn.sigmoid(h)
    y = jnp.dot(h, w2_ref[0], preferred_element_type=jnp.float32)

    rows = t * T + jax.lax.broadcasted_iota(jnp.int32, (T, 1), 0)
    mask = (rows >= off_ref[e]) & (rows < off_ref[e + 1]) & (valid > 0)
    out_ref[...] += jnp.where(mask, y * w_ref[...], 0.0)


def _route_metadata(eid, grank, counts):
    eid = eid.astype(jnp.int32)
    offsets = jnp.concatenate(
        [jnp.zeros((1,), jnp.int32), jnp.cumsum(counts).astype(jnp.int32)]
    )
    inv_idx = (offsets[eid] + grank).astype(jnp.int32)
    sort_idx = (
        jnp.zeros((N,), jnp.int32)
        .at[inv_idx]
        .set(jnp.arange(N, dtype=jnp.int32))
    )

    t = jnp.arange(TM, dtype=jnp.int32)[:, None]
    e = jnp.arange(E, dtype=jnp.int32)[None, :]
    lo = offsets[:-1][None, :]
    hi = offsets[1:][None, :]
    valid = (lo < (t + 1) * T) & (hi > t * T) & (hi > lo)

    vflat = valid.reshape(-1)
    keys = (t * E + e).reshape(-1)
    cum = jnp.cumsum(vflat.astype(jnp.int32))
    nv = cum[-1]
    pos_step = jnp.where(vflat, cum - 1, STEPS)
    scat = jnp.zeros((STEPS,), jnp.int32).at[pos_step].set(keys)
    last_key = jnp.take(scat, nv - 1)
    sidx = jnp.arange(STEPS, dtype=jnp.int32)
    key_s = jnp.where(sidx < nv, scat, last_key)
    step_t = key_s // E
    step_e = key_s % E
    step_v = (sidx < nv).astype(jnp.int32)
    return sort_idx, inv_idx, offsets, step_t, step_e, step_v


def _sc_row_gather(table, idx):
    info = plsc.get_sparse_core_info()
    nw = info.num_cores * info.num_subcores
    bpw = N // nw
    mesh = plsc.VectorSubcoreMesh(core_axis_name="c", subcore_axis_name="s")

    @functools.partial(
        pl.kernel,
        mesh=mesh,
        out_type=jax.ShapeDtypeStruct((N, D), jnp.float32),
        scratch_types=[
            pltpu.VMEM((bpw,), jnp.int32),
            pltpu.VMEM((bpw, D), jnp.float32),
            pltpu.SemaphoreType.DMA,
        ],
    )
    def gather_k(table_hbm, idx_hbm, out_hbm, idx_v, rows_v, sem):
        wid = jax.lax.axis_index("s") * info.num_cores + jax.lax.axis_index("c")
        base = wid * bpw
        pltpu.sync_copy(idx_hbm.at[pl.ds(base, bpw)], idx_v)
        pltpu.async_copy(table_hbm.at[idx_v], rows_v, sem).wait()
        pltpu.sync_copy(rows_v, out_hbm.at[pl.ds(base, bpw)])

    return gather_k(table, idx)


def kernel(x, gate_w, w1, w2):
    orig_shape = x.shape
    xf = x.reshape(-1, x.shape[-1]).astype(jnp.float32)
    gwt = jnp.zeros((D, EPAD), jnp.float32).at[:, :E].set(gate_w.T)

    eid3, grank3, cnt = pl.pallas_call(
        _gating_body,
        grid=(GM,),
        in_specs=[
            pl.BlockSpec((GT, D), lambda t: (t, 0)),
            pl.BlockSpec((D, EPAD), lambda t: (0, 0)),
        ],
        out_specs=[
            pl.BlockSpec((1, 1, GT), lambda t: (t, 0, 0)),
            pl.BlockSpec((1, 1, GT), lambda t: (t, 0, 0)),
            pl.BlockSpec((1, EPAD), lambda t: (0, 0)),
        ],
        out_shape=[
            jax.ShapeDtypeStruct((GM, 1, GT), jnp.int32),
            jax.ShapeDtypeStruct((GM, 1, GT), jnp.int32),
            jax.ShapeDtypeStruct((1, EPAD), jnp.int32),
        ],
    )(xf, gwt)
    eid = eid3.reshape(N)
    grank = grank3.reshape(N)
    counts = cnt[0, :E]

    sort_idx, inv_idx, offsets, step_t, step_e, step_v = _route_metadata(
        eid, grank, counts)

    x_sorted = _sc_row_gather(xf, sort_idx)

    grid_spec = pltpu.PrefetchScalarGridSpec(
        num_scalar_prefetch=4,
        grid=(STEPS, F2),
        in_specs=[
            pl.BlockSpec((T, D), lambda s, f, st, se, sv, off: (st[s], 0)),
            pl.BlockSpec((1, D, FS),
                         lambda s, f, st, se, sv, off: (se[s], 0, f)),
            pl.BlockSpec((1, FS, D),
                         lambda s, f, st, se, sv, off: (se[s], f, 0)),
            pl.BlockSpec((D, EPAD), lambda s, f, st, se, sv, off: (0, 0)),
        ],
        out_specs=pl.BlockSpec((T, D), lambda s, f, st, se, sv, off: (st[s], 0)),
        scratch_shapes=[pltpu.VMEM((T, 1), jnp.float32)],
    )
    out_sorted = pl.pallas_call(
        _ffn_body,
        grid_spec=grid_spec,
        out_shape=jax.ShapeDtypeStruct((N, D), jnp.float32),
        compiler_params=pltpu.CompilerParams(
            dimension_semantics=("arbitrary", "arbitrary"),
        ),
    )(step_t, step_e, step_v, offsets, x_sorted, w1, w2, gwt)

    y = _sc_row_gather(out_sorted, inv_idx)
    return y.reshape(orig_shape)


# speedup vs baseline: 1.5333x; 1.1140x over previous
import functools

import jax
import jax.numpy as jnp
from jax.experimental import pallas as pl
from jax.experimental.pallas import tpu as pltpu
from jax.experimental.pallas import tpu_sc as plsc

E = 64
D = 768
FF = 1024
N = 2048
T = 128
TM = N // T
STEPS = TM + E - 1
GT = 128
GM = N // GT
EPAD = 128
_BIG = 1 << 30


def _gating_body(x_ref, gwt_ref, eid_ref, grank_ref, cnt_ref):
    t = pl.program_id(0)

    @pl.when(t == 0)
    def _():
        cnt_ref[...] = jnp.zeros_like(cnt_ref)

    xb = x_ref[...]
    s = jnp.dot(xb, gwt_ref[...], preferred_element_type=jnp.float32)
    col = jax.lax.broadcasted_iota(jnp.int32, s.shape, 1)
    s = jnp.where(col < E, s, -jnp.inf)
    m = jnp.max(s, axis=1, keepdims=True)
    idx = jnp.min(jnp.where(s == m, col, E), axis=1)
    onehot = (col == idx[:, None]).astype(jnp.int32)
    ri = jax.lax.broadcasted_iota(jnp.int32, (GT, GT), 0)
    rj = jax.lax.broadcasted_iota(jnp.int32, (GT, GT), 1)
    same = (idx[:, None] == idx[None, :]) & (rj < ri)
    rank = jnp.sum(same.astype(jnp.int32), axis=1)
    base = jnp.sum(onehot * cnt_ref[...], axis=1)
    eid_ref[...] = idx.reshape(1, 1, GT)
    grank_ref[...] = (base + rank).reshape(1, 1, GT)
    cnt_ref[...] += jnp.sum(onehot, axis=0, keepdims=True)


F2 = 1
FS = FF // F2


def _ffn_body(st_ref, se_ref, sv_ref, off_ref,
              x_ref, w1_ref, w2_ref, gwt_ref, out_ref, w_ref):
    s = pl.program_id(0)
    f = pl.program_id(1)
    t = st_ref[s]
    e = se_ref[s]
    valid = sv_ref[s]
    prev_t = st_ref[jnp.maximum(s - 1, 0)]
    first = (f == 0) & jnp.logical_or(s == 0, prev_t != t)

    @pl.when(first)
    def _():
        out_ref[...] = jnp.zeros_like(out_ref)

    xb = x_ref[...]

    @pl.when(f == 0)
    def _():
        sc = jnp.dot(xb, gwt_ref[...], preferred_element_type=jnp.float32)
        col = jax.lax.broadcasted_iota(jnp.int32, sc.shape, 1)
        sc = jnp.where(col < E, sc, -jnp.inf)
        m = jnp.max(sc, axis=1, keepdims=True)
        w_ref[...] = 1.0 / jnp.sum(jnp.exp(sc - m), axis=1, keepdims=True)

    h = jnp.dot(xb, w1_ref[0], preferred_element_type=jnp.float32)
    h = h * jax.nn.sigmoid(h)
    y = jnp.dot(h, w2_ref[0], preferred_element_type=jnp.float32)

    rows = t * T + jax.lax.broadcasted_iota(jnp.int32, (T, 1), 0)
    mask = (rows >= off_ref[e]) & (rows < off_ref[e + 1]) & (valid > 0)
    out_ref[...] += jnp.where(mask, y * w_ref[...], 0.0)


def _meta_body(off_ref, eid_ref, grank_ref, inv_ref, smeta_ref):
    eid = eid_ref[...].reshape(GM, GT)
    grank = grank_ref[...].reshape(GM, GT)

    def body(e, carry):
        off_of, lo_vec, hi_vec = carry
        lo_e = off_ref[e]
        hi_e = off_ref[e + 1]
        lane = jax.lax.broadcasted_iota(jnp.int32, (1, EPAD), 1)
        off_of = jnp.where(eid == e, lo_e, off_of)
        lo_vec = jnp.where(lane == e, lo_e, lo_vec)
        hi_vec = jnp.where(lane == e, hi_e, hi_vec)
        return off_of, lo_vec, hi_vec

    zero_gm = jnp.zeros((GM, GT), jnp.int32)
    zero_ln = jnp.zeros((1, EPAD), jnp.int32)
    off_of, lo_vec, hi_vec = jax.lax.fori_loop(
        0, E, body, (zero_gm, zero_ln, zero_ln))
    inv_ref[...] = off_of + grank

    lane = jax.lax.broadcasted_iota(jnp.int32, (TM, EPAD), 1)
    trow = jax.lax.broadcasted_iota(jnp.int32, (TM, EPAD), 0)
    valid = ((lo_vec < (trow + 1) * T) & (hi_vec > trow * T)
             & (hi_vec > lo_vec) & (lane < E))
    vf = valid.astype(jnp.float32)

    li = jax.lax.broadcasted_iota(jnp.int32, (EPAD, EPAD), 0)
    lj = jax.lax.broadcasted_iota(jnp.int32, (EPAD, EPAD), 1)
    ltl = (li <= lj).astype(jnp.float32)
    rowpfx = jnp.dot(vf, ltl, preferred_element_type=jnp.float32)
    tot = rowpfx[:, EPAD - 1:EPAD]
    ri = jax.lax.broadcasted_iota(jnp.int32, (TM, TM), 0)
    rj = jax.lax.broadcasted_iota(jnp.int32, (TM, TM), 1)
    ltr = (rj < ri).astype(jnp.float32)
    rowoff = jnp.dot(ltr, tot, preferred_element_type=jnp.float32)
    cum = rowoff + rowpfx
    pos = jnp.where(valid, cum - 1.0, 9999.0)

    scol = jax.lax.broadcasted_iota(
        jnp.int32, (EPAD, 1), 0).astype(jnp.float32)
    lane_f = jax.lax.broadcasted_iota(
        jnp.int32, (1, EPAD), 1).astype(jnp.float32)
    st_f = jnp.zeros((EPAD, 1), jnp.float32)
    se_f = jnp.zeros((EPAD, 1), jnp.float32)
    for t in range(TM):
        p_t = (pos[t:t + 1, :] == scol).astype(jnp.float32)
        hit = jnp.sum(p_t, axis=1, keepdims=True)
        st_f = st_f + hit * jnp.float32(t)
        se_f = se_f + jnp.sum(p_t * lane_f, axis=1, keepdims=True)

    nv = jnp.sum(vf)
    svec = scol
    last_t = jnp.sum(jnp.where(svec == nv - 1.0, st_f, 0.0))
    last_e = jnp.sum(jnp.where(svec == nv - 1.0, se_f, 0.0))
    st = jnp.where(svec < nv, st_f, last_t).astype(jnp.int32)
    se = jnp.where(svec < nv, se_f, last_e).astype(jnp.int32)
    sv = (svec < nv).astype(jnp.int32)
    pad = jnp.zeros((EPAD, 5), jnp.int32)
    smeta_ref[...] = jnp.concatenate([st, se, sv, pad], axis=1)


def _sc_row_gather(table, idx):
    info = plsc.get_sparse_core_info()
    nw = info.num_cores * info.num_subcores
    bpw = N // nw
    mesh = plsc.VectorSubcoreMesh(core_axis_name="c", subcore_axis_name="s")

    @functools.partial(
        pl.kernel,
        mesh=mesh,
        out_type=jax.ShapeDtypeStruct((N, D), jnp.float32),
        scratch_types=[
            pltpu.VMEM((bpw,), jnp.int32),
            pltpu.VMEM((bpw, D), jnp.float32),
            pltpu.SemaphoreType.DMA,
        ],
    )
    def gather_k(table_hbm, idx_hbm, out_hbm, idx_v, rows_v, sem):
        wid = jax.lax.axis_index("s") * info.num_cores + jax.lax.axis_index("c")
        base = wid * bpw
        pltpu.sync_copy(idx_hbm.at[pl.ds(base, bpw)], idx_v)
        pltpu.async_copy(table_hbm.at[idx_v], rows_v, sem).wait()
        pltpu.sync_copy(rows_v, out_hbm.at[pl.ds(base, bpw)])

    return gather_k(table, idx)


def _sc_row_scatter(table, idx):
    info = plsc.get_sparse_core_info()
    nw = info.num_cores * info.num_subcores
    bpw = N // nw
    mesh = plsc.VectorSubcoreMesh(core_axis_name="c", subcore_axis_name="s")

    @functools.partial(
        pl.kernel,
        mesh=mesh,
        out_type=jax.ShapeDtypeStruct((N, D), jnp.float32),
        scratch_types=[
            pltpu.VMEM((bpw,), jnp.int32),
            pltpu.VMEM((bpw, D), jnp.float32),
            pltpu.SemaphoreType.DMA,
        ],
    )
    def scatter_k(table_hbm, idx_hbm, out_hbm, idx_v, rows_v, sem):
        wid = jax.lax.axis_index("s") * info.num_cores + jax.lax.axis_index("c")
        base = wid * bpw
        pltpu.sync_copy(idx_hbm.at[pl.ds(base, bpw)], idx_v)
        pltpu.sync_copy(table_hbm.at[pl.ds(base, bpw)], rows_v)
        pltpu.async_copy(rows_v, out_hbm.at[idx_v], sem).wait()

    return scatter_k(table, idx)


def kernel(x, gate_w, w1, w2):
    orig_shape = x.shape
    xf = x.reshape(-1, x.shape[-1]).astype(jnp.float32)
    gwt = jnp.zeros((D, EPAD), jnp.float32).at[:, :E].set(gate_w.T)

    eid3, grank3, cnt = pl.pallas_call(
        _gating_body,
        grid=(GM,),
        in_specs=[
            pl.BlockSpec((GT, D), lambda t: (t, 0)),
            pl.BlockSpec((D, EPAD), lambda t: (0, 0)),
        ],
        out_specs=[
            pl.BlockSpec((1, 1, GT), lambda t: (t, 0, 0)),
            pl.BlockSpec((1, 1, GT), lambda t: (t, 0, 0)),
            pl.BlockSpec((1, EPAD), lambda t: (0, 0)),
        ],
        out_shape=[
            jax.ShapeDtypeStruct((GM, 1, GT), jnp.int32),
            jax.ShapeDtypeStruct((GM, 1, GT), jnp.int32),
            jax.ShapeDtypeStruct((1, EPAD), jnp.int32),
        ],
    )(xf, gwt)
    counts = cnt[0, :E]
    offsets = jnp.concatenate(
        [jnp.zeros((1,), jnp.int32), jnp.cumsum(counts).astype(jnp.int32)]
    )

    inv2, smeta = pl.pallas_call(
        _meta_body,
        grid_spec=pltpu.PrefetchScalarGridSpec(
            num_scalar_prefetch=1,
            grid=(1,),
            in_specs=[
                pl.BlockSpec((GM, 1, GT), lambda i, off: (0, 0, 0)),
                pl.BlockSpec((GM, 1, GT), lambda i, off: (0, 0, 0)),
            ],
            out_specs=[
                pl.BlockSpec((GM, GT), lambda i, off: (0, 0)),
                pl.BlockSpec((EPAD, 8), lambda i, off: (0, 0)),
            ],
        ),
        out_shape=[
            jax.ShapeDtypeStruct((GM, GT), jnp.int32),
            jax.ShapeDtypeStruct((EPAD, 8), jnp.int32),
        ],
    )(offsets, eid3, grank3)
    inv_idx = inv2.reshape(N)
    step_t = smeta[:STEPS, 0]
    step_e = smeta[:STEPS, 1]
    step_v = smeta[:STEPS, 2]

    sort_idx = (jnp.zeros((N,), jnp.int32).at[inv_idx]
                .set(jnp.arange(N, dtype=jnp.int32)))
    x_sorted = _sc_row_gather(xf, sort_idx)

    grid_spec = pltpu.PrefetchScalarGridSpec(
        num_scalar_prefetch=4,
        grid=(STEPS, F2),
        in_specs=[
            pl.BlockSpec((T, D), lambda s, f, st, se, sv, off: (st[s], 0)),
            pl.BlockSpec((1, D, FS),
                         lambda s, f, st, se, sv, off: (se[s], 0, f)),
            pl.BlockSpec((1, FS, D),
                         lambda s, f, st, se, sv, off: (se[s], f, 0)),
            pl.BlockSpec((D, EPAD), lambda s, f, st, se, sv, off: (0, 0)),
        ],
        out_specs=pl.BlockSpec((T, D), lambda s, f, st, se, sv, off: (st[s], 0)),
        scratch_shapes=[pltpu.VMEM((T, 1), jnp.float32)],
    )
    out_sorted = pl.pallas_call(
        _ffn_body,
        grid_spec=grid_spec,
        out_shape=jax.ShapeDtypeStruct((N, D), jnp.float32),
        compiler_params=pltpu.CompilerParams(
            dimension_semantics=("arbitrary", "arbitrary"),
        ),
    )(step_t, step_e, step_v, offsets, x_sorted, w1, w2, gwt)

    y = _sc_row_gather(out_sorted, inv_idx)
    return y.reshape(orig_shape)


# speedup vs baseline: 1.6024x; 1.0451x over previous
import functools

import jax
import jax.numpy as jnp
from jax.experimental import pallas as pl
from jax.experimental.pallas import tpu as pltpu
from jax.experimental.pallas import tpu_sc as plsc

E = 64
D = 768
FF = 1024
N = 2048
T = 128
TM = N // T
STEPS = TM + E - 1
GT = 128
GM = N // GT
EPAD = 128
_BIG = 1 << 30


def _gating_body(x_ref, gwt_ref, eid_ref, grank_ref, cnt_ref):
    t = pl.program_id(0)

    @pl.when(t == 0)
    def _():
        cnt_ref[...] = jnp.zeros_like(cnt_ref)

    xb = x_ref[...]
    s = jnp.dot(xb, gwt_ref[...], preferred_element_type=jnp.float32)
    col = jax.lax.broadcasted_iota(jnp.int32, s.shape, 1)
    s = jnp.where(col < E, s, -jnp.inf)
    m = jnp.max(s, axis=1, keepdims=True)
    idx = jnp.min(jnp.where(s == m, col, E), axis=1)
    onehot = (col == idx[:, None]).astype(jnp.int32)
    ri = jax.lax.broadcasted_iota(jnp.int32, (GT, GT), 0)
    rj = jax.lax.broadcasted_iota(jnp.int32, (GT, GT), 1)
    same = (idx[:, None] == idx[None, :]) & (rj < ri)
    rank = jnp.sum(same.astype(jnp.int32), axis=1)
    base = jnp.sum(onehot * cnt_ref[...], axis=1)
    eid_ref[...] = idx.reshape(1, 1, GT)
    grank_ref[...] = (base + rank).reshape(1, 1, GT)
    cnt_ref[...] += jnp.sum(onehot, axis=0, keepdims=True)


F2 = 1
FS = FF // F2


def _ffn_body(st_ref, se_ref, sv_ref, off_ref,
              x_ref, w1_ref, w2_ref, gwt_ref, out_ref, w_ref):
    s = pl.program_id(0)
    f = pl.program_id(1)
    t = st_ref[s]
    e = se_ref[s]
    valid = sv_ref[s]
    prev_t = st_ref[jnp.maximum(s - 1, 0)]
    first = (f == 0) & jnp.logical_or(s == 0, prev_t != t)

    @pl.when(first)
    def _():
        out_ref[...] = jnp.zeros_like(out_ref)

    xb = x_ref[...]

    @pl.when(f == 0)
    def _():
        sc = jnp.dot(xb, gwt_ref[...], preferred_element_type=jnp.float32)
        col = jax.lax.broadcasted_iota(jnp.int32, sc.shape, 1)
        sc = jnp.where(col < E, sc, -jnp.inf)
        m = jnp.max(sc, axis=1, keepdims=True)
        w_ref[...] = 1.0 / jnp.sum(jnp.exp(sc - m), axis=1, keepdims=True)

    h = jnp.dot(xb, w1_ref[0], preferred_element_type=jnp.float32)
    h = h * jax.nn.sigmoid(h)
    y = jnp.dot(h, w2_ref[0], preferred_element_type=jnp.float32)

    rows = t * T + jax.lax.broadcasted_iota(jnp.int32, (T, 1), 0)
    mask = (rows >= off_ref[e]) & (rows < off_ref[e + 1]) & (valid > 0)
    out_ref[...] += jnp.where(mask, y * w_ref[...], 0.0)


def _meta_body(off_ref, eid_ref, grank_ref, inv_ref, smeta_ref):
    eid = eid_ref[...].reshape(GM, GT)
    grank = grank_ref[...].reshape(GM, GT)

    def body(e, carry):
        off_of, lo_vec, hi_vec = carry
        lo_e = off_ref[e]
        hi_e = off_ref[e + 1]
        lane = jax.lax.broadcasted_iota(jnp.int32, (1, EPAD), 1)
        off_of = jnp.where(eid == e, lo_e, off_of)
        lo_vec = jnp.where(lane == e, lo_e, lo_vec)
        hi_vec = jnp.where(lane == e, hi_e, hi_vec)
        return off_of, lo_vec, hi_vec

    zero_gm = jnp.zeros((GM, GT), jnp.int32)
    zero_ln = jnp.zeros((1, EPAD), jnp.int32)
    off_of, lo_vec, hi_vec = jax.lax.fori_loop(
        0, E, body, (zero_gm, zero_ln, zero_ln))
    inv_ref[...] = off_of + grank

    lane = jax.lax.broadcasted_iota(jnp.int32, (TM, EPAD), 1)
    trow = jax.lax.broadcasted_iota(jnp.int32, (TM, EPAD), 0)
    valid = ((lo_vec < (trow + 1) * T) & (hi_vec > trow * T)
             & (hi_vec > lo_vec) & (lane < E))
    vf = valid.astype(jnp.float32)

    li = jax.lax.broadcasted_iota(jnp.int32, (EPAD, EPAD), 0)
    lj = jax.lax.broadcasted_iota(jnp.int32, (EPAD, EPAD), 1)
    ltl = (li <= lj).astype(jnp.float32)
    rowpfx = jnp.dot(vf, ltl, preferred_element_type=jnp.float32)
    tot = rowpfx[:, EPAD - 1:EPAD]
    ri = jax.lax.broadcasted_iota(jnp.int32, (TM, TM), 0)
    rj = jax.lax.broadcasted_iota(jnp.int32, (TM, TM), 1)
    ltr = (rj < ri).astype(jnp.float32)
    rowoff = jnp.dot(ltr, tot, preferred_element_type=jnp.float32)
    cum = rowoff + rowpfx
    pos = jnp.where(valid, cum - 1.0, 9999.0)

    scol = jax.lax.broadcasted_iota(
        jnp.int32, (EPAD, 1), 0).astype(jnp.float32)
    lane_f = jax.lax.broadcasted_iota(
        jnp.int32, (1, EPAD), 1).astype(jnp.float32)
    st_f = jnp.zeros((EPAD, 1), jnp.float32)
    se_f = jnp.zeros((EPAD, 1), jnp.float32)
    for t in range(TM):
        p_t = (pos[t:t + 1, :] == scol).astype(jnp.float32)
        hit = jnp.sum(p_t, axis=1, keepdims=True)
        st_f = st_f + hit * jnp.float32(t)
        se_f = se_f + jnp.sum(p_t * lane_f, axis=1, keepdims=True)

    nv = jnp.sum(vf)
    svec = scol
    last_t = jnp.sum(jnp.where(svec == nv - 1.0, st_f, 0.0))
    last_e = jnp.sum(jnp.where(svec == nv - 1.0, se_f, 0.0))
    st = jnp.where(svec < nv, st_f, last_t).astype(jnp.int32)
    se = jnp.where(svec < nv, se_f, last_e).astype(jnp.int32)
    sv = (svec < nv).astype(jnp.int32)
    pad = jnp.zeros((EPAD, 5), jnp.int32)
    smeta_ref[...] = jnp.concatenate([st, se, sv, pad], axis=1)


def _sc_row_gather(table, idx):
    info = plsc.get_sparse_core_info()
    nw = info.num_cores * info.num_subcores
    bpw = N // nw
    mesh = plsc.VectorSubcoreMesh(core_axis_name="c", subcore_axis_name="s")

    @functools.partial(
        pl.kernel,
        mesh=mesh,
        out_type=jax.ShapeDtypeStruct((N, D), jnp.float32),
        scratch_types=[
            pltpu.VMEM((bpw,), jnp.int32),
            pltpu.VMEM((bpw, D), jnp.float32),
            pltpu.SemaphoreType.DMA,
        ],
    )
    def gather_k(table_hbm, idx_hbm, out_hbm, idx_v, rows_v, sem):
        wid = jax.lax.axis_index("s") * info.num_cores + jax.lax.axis_index("c")
        base = wid * bpw
        pltpu.sync_copy(idx_hbm.at[pl.ds(base, bpw)], idx_v)
        pltpu.async_copy(table_hbm.at[idx_v], rows_v, sem).wait()
        pltpu.sync_copy(rows_v, out_hbm.at[pl.ds(base, bpw)])

    return gather_k(table, idx)


def _sc_row_scatter(table, idx):
    info = plsc.get_sparse_core_info()
    nw = info.num_cores * info.num_subcores
    bpw = N // nw
    mesh = plsc.VectorSubcoreMesh(core_axis_name="c", subcore_axis_name="s")

    @functools.partial(
        pl.kernel,
        mesh=mesh,
        out_type=jax.ShapeDtypeStruct((N, D), jnp.float32),
        scratch_types=[
            pltpu.VMEM((bpw,), jnp.int32),
            pltpu.VMEM((bpw, D), jnp.float32),
            pltpu.SemaphoreType.DMA,
        ],
    )
    def scatter_k(table_hbm, idx_hbm, out_hbm, idx_v, rows_v, sem):
        wid = jax.lax.axis_index("s") * info.num_cores + jax.lax.axis_index("c")
        base = wid * bpw
        pltpu.sync_copy(idx_hbm.at[pl.ds(base, bpw)], idx_v)
        pltpu.sync_copy(table_hbm.at[pl.ds(base, bpw)], rows_v)
        pltpu.async_copy(rows_v, out_hbm.at[idx_v], sem).wait()

    return scatter_k(table, idx)


def kernel(x, gate_w, w1, w2):
    orig_shape = x.shape
    xf = x.reshape(-1, x.shape[-1]).astype(jnp.float32)
    gwt = jnp.zeros((D, EPAD), jnp.float32).at[:, :E].set(gate_w.T)

    eid3, grank3, cnt = pl.pallas_call(
        _gating_body,
        grid=(GM,),
        in_specs=[
            pl.BlockSpec((GT, D), lambda t: (t, 0)),
            pl.BlockSpec((D, EPAD), lambda t: (0, 0)),
        ],
        out_specs=[
            pl.BlockSpec((1, 1, GT), lambda t: (t, 0, 0)),
            pl.BlockSpec((1, 1, GT), lambda t: (t, 0, 0)),
            pl.BlockSpec((1, EPAD), lambda t: (0, 0)),
        ],
        out_shape=[
            jax.ShapeDtypeStruct((GM, 1, GT), jnp.int32),
            jax.ShapeDtypeStruct((GM, 1, GT), jnp.int32),
            jax.ShapeDtypeStruct((1, EPAD), jnp.int32),
        ],
    )(xf, gwt)
    counts = cnt[0, :E]
    offsets = jnp.concatenate(
        [jnp.zeros((1,), jnp.int32), jnp.cumsum(counts).astype(jnp.int32)]
    )

    inv2, smeta = pl.pallas_call(
        _meta_body,
        grid_spec=pltpu.PrefetchScalarGridSpec(
            num_scalar_prefetch=1,
            grid=(1,),
            in_specs=[
                pl.BlockSpec((GM, 1, GT), lambda i, off: (0, 0, 0)),
                pl.BlockSpec((GM, 1, GT), lambda i, off: (0, 0, 0)),
            ],
            out_specs=[
                pl.BlockSpec((GM, GT), lambda i, off: (0, 0)),
                pl.BlockSpec((EPAD, 8), lambda i, off: (0, 0)),
            ],
        ),
        out_shape=[
            jax.ShapeDtypeStruct((GM, GT), jnp.int32),
            jax.ShapeDtypeStruct((EPAD, 8), jnp.int32),
        ],
    )(offsets, eid3, grank3)
    inv_idx = inv2.reshape(N)
    step_t = smeta[:STEPS, 0]
    step_e = smeta[:STEPS, 1]
    step_v = smeta[:STEPS, 2]

    x_sorted = _sc_row_scatter(xf, inv_idx)

    grid_spec = pltpu.PrefetchScalarGridSpec(
        num_scalar_prefetch=4,
        grid=(STEPS, F2),
        in_specs=[
            pl.BlockSpec((T, D), lambda s, f, st, se, sv, off: (st[s], 0)),
            pl.BlockSpec((1, D, FS),
                         lambda s, f, st, se, sv, off: (se[s], 0, f)),
            pl.BlockSpec((1, FS, D),
                         lambda s, f, st, se, sv, off: (se[s], f, 0)),
            pl.BlockSpec((D, EPAD), lambda s, f, st, se, sv, off: (0, 0)),
        ],
        out_specs=pl.BlockSpec((T, D), lambda s, f, st, se, sv, off: (st[s], 0)),
        scratch_shapes=[pltpu.VMEM((T, 1), jnp.float32)],
    )
    out_sorted = pl.pallas_call(
        _ffn_body,
        grid_spec=grid_spec,
        out_shape=jax.ShapeDtypeStruct((N, D), jnp.float32),
        compiler_params=pltpu.CompilerParams(
            dimension_semantics=("arbitrary", "arbitrary"),
        ),
    )(step_t, step_e, step_v, offsets, x_sorted, w1, w2, gwt)

    y = _sc_row_gather(out_sorted, inv_idx)
    return y.reshape(orig_shape)


# speedup vs baseline: 1.6688x; 1.0414x over previous
import functools

import jax
import jax.numpy as jnp
from jax.experimental import pallas as pl
from jax.experimental.pallas import tpu as pltpu
from jax.experimental.pallas import tpu_sc as plsc

E = 64
D = 768
FF = 1024
N = 2048
T = 128
TM = N // T
STEPS = TM + E - 1
GT = 128
GM = N // GT
EPAD = 128
_BIG = 1 << 30


def _gating_body(x_ref, gwt_ref, eid_ref, grank_ref, cnt_ref):
    xb = x_ref[...]
    s = jnp.dot(xb, gwt_ref[...], preferred_element_type=jnp.float32)
    col = jax.lax.broadcasted_iota(jnp.int32, s.shape, 1)
    s = jnp.where(col < E, s, -jnp.inf)
    m = jnp.max(s, axis=1, keepdims=True)
    idx = jnp.min(jnp.where(s == m, col, E), axis=1)
    onehot = (col == idx[:, None]).astype(jnp.int32)
    csum = onehot
    k = 1
    while k < N:
        csum = csum + jnp.concatenate(
            [jnp.zeros((k, EPAD), jnp.int32), csum[:N - k]], axis=0)
        k *= 2
    grank = jnp.sum(onehot * csum, axis=1) - 1
    eid_ref[...] = idx.reshape(GM, 1, GT)
    grank_ref[...] = grank.reshape(GM, 1, GT)
    cnt_ref[...] = csum[N - 1:N, :]


F2 = 1
FS = FF // F2


def _ffn_body(st_ref, se_ref, sv_ref, off_ref,
              x_ref, w1_ref, w2_ref, gwt_ref, out_ref, w_ref):
    s = pl.program_id(0)
    f = pl.program_id(1)
    t = st_ref[s]
    e = se_ref[s]
    valid = sv_ref[s]
    prev_t = st_ref[jnp.maximum(s - 1, 0)]
    first = (f == 0) & jnp.logical_or(s == 0, prev_t != t)

    @pl.when(first)
    def _():
        out_ref[...] = jnp.zeros_like(out_ref)

    xb = x_ref[...]

    @pl.when(f == 0)
    def _():
        sc = jnp.dot(xb, gwt_ref[...], preferred_element_type=jnp.float32)
        col = jax.lax.broadcasted_iota(jnp.int32, sc.shape, 1)
        sc = jnp.where(col < E, sc, -jnp.inf)
        m = jnp.max(sc, axis=1, keepdims=True)
        w_ref[...] = 1.0 / jnp.sum(jnp.exp(sc - m), axis=1, keepdims=True)

    h = jnp.dot(xb, w1_ref[0], preferred_element_type=jnp.float32)
    h = h * jax.nn.sigmoid(h)
    y = jnp.dot(h, w2_ref[0], preferred_element_type=jnp.float32)

    rows = t * T + jax.lax.broadcasted_iota(jnp.int32, (T, 1), 0)
    mask = (rows >= off_ref[e]) & (rows < off_ref[e + 1]) & (valid > 0)
    out_ref[...] += jnp.where(mask, y * w_ref[...], 0.0)


def _meta_body(off_ref, eid_ref, grank_ref, inv_ref, smeta_ref):
    eid = eid_ref[...].reshape(GM, GT)
    grank = grank_ref[...].reshape(GM, GT)

    def body(e, carry):
        off_of, lo_vec, hi_vec = carry
        lo_e = off_ref[e]
        hi_e = off_ref[e + 1]
        lane = jax.lax.broadcasted_iota(jnp.int32, (1, EPAD), 1)
        off_of = jnp.where(eid == e, lo_e, off_of)
        lo_vec = jnp.where(lane == e, lo_e, lo_vec)
        hi_vec = jnp.where(lane == e, hi_e, hi_vec)
        return off_of, lo_vec, hi_vec

    zero_gm = jnp.zeros((GM, GT), jnp.int32)
    zero_ln = jnp.zeros((1, EPAD), jnp.int32)
    off_of, lo_vec, hi_vec = jax.lax.fori_loop(
        0, E, body, (zero_gm, zero_ln, zero_ln))
    inv_ref[...] = off_of + grank

    lane = jax.lax.broadcasted_iota(jnp.int32, (TM, EPAD), 1)
    trow = jax.lax.broadcasted_iota(jnp.int32, (TM, EPAD), 0)
    valid = ((lo_vec < (trow + 1) * T) & (hi_vec > trow * T)
             & (hi_vec > lo_vec) & (lane < E))
    vf = valid.astype(jnp.float32)

    li = jax.lax.broadcasted_iota(jnp.int32, (EPAD, EPAD), 0)
    lj = jax.lax.broadcasted_iota(jnp.int32, (EPAD, EPAD), 1)
    ltl = (li <= lj).astype(jnp.float32)
    rowpfx = jnp.dot(vf, ltl, preferred_element_type=jnp.float32)
    tot = rowpfx[:, EPAD - 1:EPAD]
    ri = jax.lax.broadcasted_iota(jnp.int32, (TM, TM), 0)
    rj = jax.lax.broadcasted_iota(jnp.int32, (TM, TM), 1)
    ltr = (rj < ri).astype(jnp.float32)
    rowoff = jnp.dot(ltr, tot, preferred_element_type=jnp.float32)
    cum = rowoff + rowpfx
    pos = jnp.where(valid, cum - 1.0, 9999.0)

    scol = jax.lax.broadcasted_iota(
        jnp.int32, (EPAD, 1), 0).astype(jnp.float32)
    lane_f = jax.lax.broadcasted_iota(
        jnp.int32, (1, EPAD), 1).astype(jnp.float32)
    st_f = jnp.zeros((EPAD, 1), jnp.float32)
    se_f = jnp.zeros((EPAD, 1), jnp.float32)
    for t in range(TM):
        p_t = (pos[t:t + 1, :] == scol).astype(jnp.float32)
        hit = jnp.sum(p_t, axis=1, keepdims=True)
        st_f = st_f + hit * jnp.float32(t)
        se_f = se_f + jnp.sum(p_t * lane_f, axis=1, keepdims=True)

    nv = jnp.sum(vf)
    svec = scol
    last_t = jnp.sum(jnp.where(svec == nv - 1.0, st_f, 0.0))
    last_e = jnp.sum(jnp.where(svec == nv - 1.0, se_f, 0.0))
    st = jnp.where(svec < nv, st_f, last_t).astype(jnp.int32)
    se = jnp.where(svec < nv, se_f, last_e).astype(jnp.int32)
    sv = (svec < nv).astype(jnp.int32)
    pad = jnp.zeros((EPAD, 5), jnp.int32)
    smeta_ref[...] = jnp.concatenate([st, se, sv, pad], axis=1)


def _sc_row_gather(table, idx):
    info = plsc.get_sparse_core_info()
    nw = info.num_cores * info.num_subcores
    bpw = N // nw
    mesh = plsc.VectorSubcoreMesh(core_axis_name="c", subcore_axis_name="s")

    @functools.partial(
        pl.kernel,
        mesh=mesh,
        out_type=jax.ShapeDtypeStruct((N, D), jnp.float32),
        scratch_types=[
            pltpu.VMEM((bpw,), jnp.int32),
            pltpu.VMEM((bpw, D), jnp.float32),
            pltpu.SemaphoreType.DMA,
        ],
    )
    def gather_k(table_hbm, idx_hbm, out_hbm, idx_v, rows_v, sem):
        wid = jax.lax.axis_index("s") * info.num_cores + jax.lax.axis_index("c")
        base = wid * bpw
        pltpu.sync_copy(idx_hbm.at[pl.ds(base, bpw)], idx_v)
        pltpu.async_copy(table_hbm.at[idx_v], rows_v, sem).wait()
        pltpu.sync_copy(rows_v, out_hbm.at[pl.ds(base, bpw)])

    return gather_k(table, idx)


def _sc_row_scatter(table, idx):
    info = plsc.get_sparse_core_info()
    nw = info.num_cores * info.num_subcores
    bpw = N // nw
    mesh = plsc.VectorSubcoreMesh(core_axis_name="c", subcore_axis_name="s")

    @functools.partial(
        pl.kernel,
        mesh=mesh,
        out_type=jax.ShapeDtypeStruct((N, D), jnp.float32),
        scratch_types=[
            pltpu.VMEM((bpw,), jnp.int32),
            pltpu.VMEM((bpw, D), jnp.float32),
            pltpu.SemaphoreType.DMA,
        ],
    )
    def scatter_k(table_hbm, idx_hbm, out_hbm, idx_v, rows_v, sem):
        wid = jax.lax.axis_index("s") * info.num_cores + jax.lax.axis_index("c")
        base = wid * bpw
        pltpu.sync_copy(idx_hbm.at[pl.ds(base, bpw)], idx_v)
        pltpu.sync_copy(table_hbm.at[pl.ds(base, bpw)], rows_v)
        pltpu.async_copy(rows_v, out_hbm.at[idx_v], sem).wait()

    return scatter_k(table, idx)


def kernel(x, gate_w, w1, w2):
    orig_shape = x.shape
    xf = x.reshape(-1, x.shape[-1]).astype(jnp.float32)
    gwt = jnp.zeros((D, EPAD), jnp.float32).at[:, :E].set(gate_w.T)

    eid3, grank3, cnt = pl.pallas_call(
        _gating_body,
        out_shape=[
            jax.ShapeDtypeStruct((GM, 1, GT), jnp.int32),
            jax.ShapeDtypeStruct((GM, 1, GT), jnp.int32),
            jax.ShapeDtypeStruct((1, EPAD), jnp.int32),
        ],
    )(xf, gwt)
    counts = cnt[0, :E]
    offsets = jnp.concatenate(
        [jnp.zeros((1,), jnp.int32), jnp.cumsum(counts).astype(jnp.int32)]
    )

    inv2, smeta = pl.pallas_call(
        _meta_body,
        grid_spec=pltpu.PrefetchScalarGridSpec(
            num_scalar_prefetch=1,
            grid=(1,),
            in_specs=[
                pl.BlockSpec((GM, 1, GT), lambda i, off: (0, 0, 0)),
                pl.BlockSpec((GM, 1, GT), lambda i, off: (0, 0, 0)),
            ],
            out_specs=[
                pl.BlockSpec((GM, GT), lambda i, off: (0, 0)),
                pl.BlockSpec((EPAD, 8), lambda i, off: (0, 0)),
            ],
        ),
        out_shape=[
            jax.ShapeDtypeStruct((GM, GT), jnp.int32),
            jax.ShapeDtypeStruct((EPAD, 8), jnp.int32),
        ],
    )(offsets, eid3, grank3)
    inv_idx = inv2.reshape(N)
    step_t = smeta[:STEPS, 0]
    step_e = smeta[:STEPS, 1]
    step_v = smeta[:STEPS, 2]

    x_sorted = _sc_row_scatter(xf, inv_idx)

    grid_spec = pltpu.PrefetchScalarGridSpec(
        num_scalar_prefetch=4,
        grid=(STEPS, F2),
        in_specs=[
            pl.BlockSpec((T, D), lambda s, f, st, se, sv, off: (st[s], 0)),
            pl.BlockSpec((1, D, FS),
                         lambda s, f, st, se, sv, off: (se[s], 0, f)),
            pl.BlockSpec((1, FS, D),
                         lambda s, f, st, se, sv, off: (se[s], f, 0)),
            pl.BlockSpec((D, EPAD), lambda s, f, st, se, sv, off: (0, 0)),
        ],
        out_specs=pl.BlockSpec((T, D), lambda s, f, st, se, sv, off: (st[s], 0)),
        scratch_shapes=[pltpu.VMEM((T, 1), jnp.float32)],
    )
    out_sorted = pl.pallas_call(
        _ffn_body,
        grid_spec=grid_spec,
        out_shape=jax.ShapeDtypeStruct((N, D), jnp.float32),
        compiler_params=pltpu.CompilerParams(
            dimension_semantics=("arbitrary", "arbitrary"),
        ),
    )(step_t, step_e, step_v, offsets, x_sorted, w1, w2, gwt)

    y = _sc_row_gather(out_sorted, inv_idx)
    return y.reshape(orig_shape)
